# Initial kernel scaffold; baseline (speedup 1.0000x reference)
import jax
import jax.numpy as jnp
from jax.experimental import pallas as pl


def kernel(nodes_0, nodes_1, nodes_2, nodes_3, counts_0, counts_1, W_proj, b_proj, W30, b30, W31, b31, W32, b32, W20, b20, W21, b21, W10, b10, W00, b00, W, b):
    raise NotImplementedError("write your pallas kernel here")


# speedup vs baseline: 1.7126x; 1.7126x over previous
import jax
import jax.numpy as jnp
from jax.experimental import pallas as pl
from jax.experimental.pallas import tpu as pltpu

_B = 128
_F = 256
_C0 = 256
_C1 = 64
_N1 = 16
_TB = 16
_STEPS = _B // _TB
_EPS = 1e-5
_F32 = jnp.float32


def _chunk_mask(dd, k):
    l = dd // k
    j = jax.lax.broadcasted_iota(jnp.int32, (dd, k), 0)
    kk = jax.lax.broadcasted_iota(jnp.int32, (dd, k), 1)
    return (j // l == kk).astype(_F32)


def _acc_stats(t, k, s_ref, q_ref):
    dd = t.shape[-1]
    s = jnp.sum(t, axis=0)
    q = jnp.sum(t * t, axis=0)
    if k == dd:
        s_ref[...] += s
        q_ref[...] += q
    else:
        m = _chunk_mask(dd, k)
        s_ref[...] += jnp.dot(s, m, preferred_element_type=_F32)
        q_ref[...] += jnp.dot(q, m, preferred_element_type=_F32)


def _stage1(n0, w3, b3, n1, w2, b2, n2, w1, b1, n3, w0, b0,
            t1_o, ty_o, tz_o, tw_o, s3, q3, s2, q2, s1, q1, s0, q0):
    i = pl.program_id(0)

    @pl.when(i == 0)
    def _():
        for r in (s3, q3, s2, q2, s1, q1, s0, q0):
            r[...] = jnp.zeros_like(r)

    def hop(n_ref, w_ref, b_ref, o_ref, l):
        x = n_ref[...].reshape(_TB * l, _F)
        t = jnp.dot(x, w_ref[...], preferred_element_type=_F32) + b_ref[...]
        t = t.reshape(_TB, l, t.shape[-1])
        o_ref[...] = t
        return t

    _acc_stats(hop(n0, w3, b3, t1_o, _C0), 1, s3, q3)
    _acc_stats(hop(n1, w2, b2, ty_o, _C1), 4, s2, q2)
    _acc_stats(hop(n2, w1, b1, tz_o, _N1), 16, s1, q1)
    _acc_stats(hop(n3, w0, b0, tw_o, 1), 256, s0, q0)


def _stage2(t1_ref, a_ref, sh_ref, w_ref, b_ref, out_ref, s_ref, q_ref):
    i = pl.program_id(0)

    @pl.when(i == 0)
    def _():
        s_ref[...] = jnp.zeros_like(s_ref)
        q_ref[...] = jnp.zeros_like(q_ref)

    p = jnp.maximum(t1_ref[...] * a_ref[...][None] - sh_ref[...][None], 0.0)
    t = jnp.dot(p.reshape(_TB * _C0, 2 * 128), w_ref[...],
                preferred_element_type=_F32) + b_ref[...]
    t = t.reshape(_TB, _C0, 512)
    out_ref[...] = t[:, :_C1, :]
    _acc_stats(t, 2, s_ref, q_ref)


def _stage3(t3_ref, a3, sh3, w32, b32r, ty1_ref, a2, sh2, w21, b21r,
            t5_o, s5, q5, ty2_o, sy, qy):
    i = pl.program_id(0)

    @pl.when(i == 0)
    def _():
        for r in (s5, q5, sy, qy):
            r[...] = jnp.zeros_like(r)

    p = jnp.maximum(t3_ref[...] * a3[...][None] - sh3[...][None], 0.0)
    t5 = jnp.dot(p.reshape(_TB * _C1, 512), w32[...],
                 preferred_element_type=_F32) + b32r[...]
    t5 = t5.reshape(_TB, _C1, 1024)
    t5_o[...] = t5
    _acc_stats(t5, 16, s5, q5)

    p2 = jnp.maximum(ty1_ref[...] * a2[...][None] - sh2[...][None], 0.0)
    ty2 = jnp.dot(p2.reshape(_TB * _C1, 256), w21[...],
                  preferred_element_type=_F32) + b21r[...]
    ty2 = ty2.reshape(_TB, _C1, 512)
    ty2_o[...] = ty2
    _acc_stats(ty2, 8, sy, qy)


def _stage4(t5_ref, a5, c5, ty2_ref, ay, cy, tz_ref, az, cz,
            tw_ref, a0, c0, w_ref, b_ref, out_ref):
    h3 = jnp.mean(t5_ref[...] * a5[...][None], axis=1) - c5[...]
    h2 = jnp.mean(ty2_ref[...] * ay[...][None], axis=1) - cy[...]
    h1 = jnp.mean(tz_ref[...] * az[...][None], axis=1) - cz[...]
    h0 = jnp.mean(tw_ref[...] * a0[...][None], axis=1) - c0[...]
    h = jnp.concatenate([h0, h1, h2, h3], axis=1)
    out_ref[...] = jnp.dot(h, w_ref[...], preferred_element_type=_F32) + b_ref[...]


def _bn_ab(s, q, l, cnt=None, rows=None):
    n = _B * l
    mean = s / n
    var = q / n - mean * mean
    rstd = jax.lax.rsqrt(var + _EPS)
    mean_f = jnp.repeat(mean, l, axis=1)
    rstd_f = jnp.repeat(rstd, l, axis=1)
    if rows is not None:
        mean_f = mean_f[:rows]
        rstd_f = rstd_f[:rows]
    a = rstd_f if cnt is None else rstd_f / cnt[:, None]
    return a, mean_f * a


def _bspec(l, d):
    return pl.BlockSpec((_TB, l, d), lambda i: (i, 0, 0))


def _cspec(r, c):
    return pl.BlockSpec((r, c), lambda i: (0, 0))


_CP = pltpu.CompilerParams(dimension_semantics=("arbitrary",))


def kernel(nodes_0, nodes_1, nodes_2, nodes_3, counts_0, counts_1,
           W_proj, b_proj, W30, b30, W31, b31, W32, b32,
           W20, b20, W21, b21, W10, b10, W00, b00, W, b):
    f = _F32
    cnt0 = counts_0[0].astype(f)
    cnt1 = counts_1[0].astype(f)

    wc3, bc3 = W_proj @ W30, (b_proj @ W30 + b30)[None, :]
    wc2, bc2 = W_proj @ W20, (b_proj @ W20 + b20)[None, :]
    wc1, bc1 = W_proj @ W10, (b_proj @ W10 + b10)[None, :]
    wc0, bc0 = W_proj @ W00, (b_proj @ W00 + b00)[None, :]

    s1_out = pl.pallas_call(
        _stage1,
        grid=(_STEPS,),
        in_specs=[
            _bspec(_C0, _F), _cspec(_F, 256), _cspec(1, 256),
            _bspec(_C1, _F), _cspec(_F, 256), _cspec(1, 256),
            _bspec(_N1, _F), _cspec(_F, 256), _cspec(1, 256),
            _bspec(1, _F), _cspec(_F, 256), _cspec(1, 256),
        ],
        out_specs=[
            _bspec(_C0, 256), _bspec(_C1, 256), _bspec(_N1, 256), _bspec(1, 256),
            _cspec(_C0, 1), _cspec(_C0, 1),
            _cspec(_C1, 4), _cspec(_C1, 4),
            _cspec(_N1, 16), _cspec(_N1, 16),
            _cspec(1, 256), _cspec(1, 256),
        ],
        out_shape=[
            jax.ShapeDtypeStruct((_B, _C0, 256), f),
            jax.ShapeDtypeStruct((_B, _C1, 256), f),
            jax.ShapeDtypeStruct((_B, _N1, 256), f),
            jax.ShapeDtypeStruct((_B, 1, 256), f),
            jax.ShapeDtypeStruct((_C0, 1), f), jax.ShapeDtypeStruct((_C0, 1), f),
            jax.ShapeDtypeStruct((_C1, 4), f), jax.ShapeDtypeStruct((_C1, 4), f),
            jax.ShapeDtypeStruct((_N1, 16), f), jax.ShapeDtypeStruct((_N1, 16), f),
            jax.ShapeDtypeStruct((1, 256), f), jax.ShapeDtypeStruct((1, 256), f),
        ],
        compiler_params=_CP,
    )(nodes_0, wc3, bc3, nodes_1, wc2, bc2, nodes_2, wc1, bc1, nodes_3, wc0, bc0)
    t1, ty1, tz1, tw1, s3, q3, s2, q2, sz, qz, s0, q0 = s1_out

    a1, sh1 = _bn_ab(s3, q3, _C0, cnt=cnt0)

    t3r, s31, q31 = pl.pallas_call(
        _stage2,
        grid=(_STEPS,),
        in_specs=[_bspec(_C0, 256), _cspec(_C0, 256), _cspec(_C0, 256),
                  _cspec(256, 512), _cspec(1, 512)],
        out_specs=[_bspec(_C1, 512), _cspec(_C0, 2), _cspec(_C0, 2)],
        out_shape=[
            jax.ShapeDtypeStruct((_B, _C1, 512), f),
            jax.ShapeDtypeStruct((_C0, 2), f), jax.ShapeDtypeStruct((_C0, 2), f),
        ],
        compiler_params=_CP,
    )(t1, a1, sh1, W31, b31[None, :])

    a3, sh3 = _bn_ab(s31, q31, _C0, cnt=cnt1, rows=_C1)
    a2, sh2 = _bn_ab(s2, q2, _C1, cnt=cnt1)

    t5, s5, q5, ty2, sy, qy = pl.pallas_call(
        _stage3,
        grid=(_STEPS,),
        in_specs=[_bspec(_C1, 512), _cspec(_C1, 512), _cspec(_C1, 512),
                  _cspec(512, 1024), _cspec(1, 1024),
                  _bspec(_C1, 256), _cspec(_C1, 256), _cspec(_C1, 256),
                  _cspec(256, 512), _cspec(1, 512)],
        out_specs=[_bspec(_C1, 1024), _cspec(_C1, 16), _cspec(_C1, 16),
                   _bspec(_C1, 512), _cspec(_C1, 8), _cspec(_C1, 8)],
        out_shape=[
            jax.ShapeDtypeStruct((_B, _C1, 1024), f),
            jax.ShapeDtypeStruct((_C1, 16), f), jax.ShapeDtypeStruct((_C1, 16), f),
            jax.ShapeDtypeStruct((_B, _C1, 512), f),
            jax.ShapeDtypeStruct((_C1, 8), f), jax.ShapeDtypeStruct((_C1, 8), f),
        ],
        compiler_params=_CP,
    )(t3r, a3, sh3, W32, b32[None, :], ty1, a2, sh2, W21, b21[None, :])

    a5, sh5 = _bn_ab(s5, q5, _C1)
    c5 = jnp.mean(sh5, axis=0, keepdims=True)
    ayt, shy = _bn_ab(sy, qy, _C1)
    cyt = jnp.mean(shy, axis=0, keepdims=True)
    azt, shz = _bn_ab(sz, qz, _N1)
    czt = jnp.mean(shz, axis=0, keepdims=True)
    a0t, sh0 = _bn_ab(s0, q0, 1)
    c0t = sh0

    out2d = pl.pallas_call(
        _stage4,
        grid=(_STEPS,),
        in_specs=[_bspec(_C1, 1024), _cspec(_C1, 1024), _cspec(1, 1024),
                  _bspec(_C1, 512), _cspec(_C1, 512), _cspec(1, 512),
                  _bspec(_N1, 256), _cspec(_N1, 256), _cspec(1, 256),
                  _bspec(1, 256), _cspec(1, 256), _cspec(1, 256),
                  _cspec(2048, 2048), _cspec(1, 2048)],
        out_specs=pl.BlockSpec((_TB, 2048), lambda i: (i, 0)),
        out_shape=jax.ShapeDtypeStruct((_B, 2048), f),
        compiler_params=_CP,
    )(t5, a5, c5, ty2, ayt, cyt, tz1, azt, czt, tw1, a0t, c0t, W, b[None, :])

    return out2d.reshape(_B, 2048, 1)


# speedup vs baseline: 1.9321x; 1.1282x over previous
import jax
import jax.numpy as jnp
from jax.experimental import pallas as pl
from jax.experimental.pallas import tpu as pltpu

_B = 128
_F = 256
_C0 = 256
_C1 = 64
_N1 = 16
_TB = 16
_STEPS = _B // _TB
_EPS = 1e-5
_F32 = jnp.float32
_BF = jnp.bfloat16


def _chunk_mask(dd, k):
    l = dd // k
    j = jax.lax.broadcasted_iota(jnp.int32, (dd, k), 0)
    kk = jax.lax.broadcasted_iota(jnp.int32, (dd, k), 1)
    return (j // l == kk).astype(_F32)


def _acc_stats(t, k, s_ref, q_ref):
    dd = t.shape[-1]
    s = jnp.sum(t, axis=0)
    q = jnp.sum(t * t, axis=0)
    if k == dd:
        s_ref[...] += s
        q_ref[...] += q
    else:
        m = _chunk_mask(dd, k)
        s_ref[...] += jnp.dot(s, m, preferred_element_type=_F32)
        q_ref[...] += jnp.dot(q, m, preferred_element_type=_F32)


def _stage1(n0, w3, b3, n1, w2, b2, n2, w1, b1, n3, w0, b0,
            t1_o, ty_o, tz_o, tw_o, s3, q3, s2, q2, s1, q1, s0, q0):
    i = pl.program_id(0)

    @pl.when(i == 0)
    def _():
        for r in (s3, q3, s2, q2, s1, q1, s0, q0):
            r[...] = jnp.zeros_like(r)

    def hop(n_ref, w_ref, b_ref, o_ref, l):
        x = n_ref[...].reshape(_TB * l, _F)
        t = jnp.dot(x, w_ref[...], preferred_element_type=_F32) + b_ref[...]
        t = t.reshape(_TB, l, t.shape[-1])
        o_ref[...] = t.astype(o_ref.dtype)
        return t

    _acc_stats(hop(n0, w3, b3, t1_o, _C0), 1, s3, q3)
    _acc_stats(hop(n1, w2, b2, ty_o, _C1), 4, s2, q2)
    _acc_stats(hop(n2, w1, b1, tz_o, _N1), 16, s1, q1)
    _acc_stats(hop(n3, w0, b0, tw_o, 1), 256, s0, q0)


def _stage2(t1_ref, a_ref, sh_ref, w_ref, b_ref, out_ref, s_ref, q_ref):
    i = pl.program_id(0)

    @pl.when(i == 0)
    def _():
        s_ref[...] = jnp.zeros_like(s_ref)
        q_ref[...] = jnp.zeros_like(q_ref)

    x = t1_ref[...].astype(_F32)
    p = jnp.maximum(x * a_ref[...][None] - sh_ref[...][None], 0.0)
    t = jnp.dot(p.reshape(_TB * _C0, 2 * 128), w_ref[...],
                preferred_element_type=_F32) + b_ref[...]
    t = t.reshape(_TB, _C0, 512)
    out_ref[...] = t[:, :_C1, :].astype(out_ref.dtype)
    _acc_stats(t, 2, s_ref, q_ref)


def _stage3(t3_ref, a3, sh3, w32, b32r, ty1_ref, a2, sh2, w21, b21r,
            t5_o, s5, q5, ty2_o, sy, qy):
    i = pl.program_id(0)

    @pl.when(i == 0)
    def _():
        for r in (s5, q5, sy, qy):
            r[...] = jnp.zeros_like(r)

    p = jnp.maximum(t3_ref[...].astype(_F32) * a3[...][None] - sh3[...][None], 0.0)
    t5 = jnp.dot(p.reshape(_TB * _C1, 512), w32[...],
                 preferred_element_type=_F32) + b32r[...]
    t5 = t5.reshape(_TB, _C1, 1024)
    t5_o[...] = t5.astype(t5_o.dtype)
    _acc_stats(t5, 16, s5, q5)

    p2 = jnp.maximum(ty1_ref[...].astype(_F32) * a2[...][None] - sh2[...][None], 0.0)
    ty2 = jnp.dot(p2.reshape(_TB * _C1, 256), w21[...],
                  preferred_element_type=_F32) + b21r[...]
    ty2 = ty2.reshape(_TB, _C1, 512)
    ty2_o[...] = ty2.astype(ty2_o.dtype)
    _acc_stats(ty2, 8, sy, qy)


def _stage4(t5_ref, a5, c5, ty2_ref, ay, cy, tz_ref, az, cz,
            tw_ref, a0, c0, w_ref, b_ref, out_ref):
    h3 = jnp.mean(t5_ref[...].astype(_F32) * a5[...][None], axis=1) - c5[...]
    h2 = jnp.mean(ty2_ref[...].astype(_F32) * ay[...][None], axis=1) - cy[...]
    h1 = jnp.mean(tz_ref[...] * az[...][None], axis=1) - cz[...]
    h0 = jnp.mean(tw_ref[...] * a0[...][None], axis=1) - c0[...]
    h = jnp.concatenate([h0, h1, h2, h3], axis=1)
    out_ref[...] = jnp.dot(h, w_ref[...], preferred_element_type=_F32) + b_ref[...]


def _bn_ab(s, q, l, cnt=None, rows=None):
    n = _B * l
    mean = s / n
    var = q / n - mean * mean
    rstd = jax.lax.rsqrt(var + _EPS)
    mean_f = jnp.repeat(mean, l, axis=1)
    rstd_f = jnp.repeat(rstd, l, axis=1)
    if rows is not None:
        mean_f = mean_f[:rows]
        rstd_f = rstd_f[:rows]
    a = rstd_f if cnt is None else rstd_f / cnt[:, None]
    return a, mean_f * a


def _bspec(l, d):
    return pl.BlockSpec((_TB, l, d), lambda i: (i, 0, 0))


def _cspec(r, c):
    return pl.BlockSpec((r, c), lambda i: (0, 0))


_CP = pltpu.CompilerParams(dimension_semantics=("arbitrary",))


def kernel(nodes_0, nodes_1, nodes_2, nodes_3, counts_0, counts_1,
           W_proj, b_proj, W30, b30, W31, b31, W32, b32,
           W20, b20, W21, b21, W10, b10, W00, b00, W, b):
    f = _F32
    cnt0 = counts_0[0].astype(f)
    cnt1 = counts_1[0].astype(f)

    wc3, bc3 = W_proj @ W30, (b_proj @ W30 + b30)[None, :]
    wc2, bc2 = W_proj @ W20, (b_proj @ W20 + b20)[None, :]
    wc1, bc1 = W_proj @ W10, (b_proj @ W10 + b10)[None, :]
    wc0, bc0 = W_proj @ W00, (b_proj @ W00 + b00)[None, :]

    s1_out = pl.pallas_call(
        _stage1,
        grid=(_STEPS,),
        in_specs=[
            _bspec(_C0, _F), _cspec(_F, 256), _cspec(1, 256),
            _bspec(_C1, _F), _cspec(_F, 256), _cspec(1, 256),
            _bspec(_N1, _F), _cspec(_F, 256), _cspec(1, 256),
            _bspec(1, _F), _cspec(_F, 256), _cspec(1, 256),
        ],
        out_specs=[
            _bspec(_C0, 256), _bspec(_C1, 256), _bspec(_N1, 256), _bspec(1, 256),
            _cspec(_C0, 1), _cspec(_C0, 1),
            _cspec(_C1, 4), _cspec(_C1, 4),
            _cspec(_N1, 16), _cspec(_N1, 16),
            _cspec(1, 256), _cspec(1, 256),
        ],
        out_shape=[
            jax.ShapeDtypeStruct((_B, _C0, 256), _BF),
            jax.ShapeDtypeStruct((_B, _C1, 256), _BF),
            jax.ShapeDtypeStruct((_B, _N1, 256), f),
            jax.ShapeDtypeStruct((_B, 1, 256), f),
            jax.ShapeDtypeStruct((_C0, 1), f), jax.ShapeDtypeStruct((_C0, 1), f),
            jax.ShapeDtypeStruct((_C1, 4), f), jax.ShapeDtypeStruct((_C1, 4), f),
            jax.ShapeDtypeStruct((_N1, 16), f), jax.ShapeDtypeStruct((_N1, 16), f),
            jax.ShapeDtypeStruct((1, 256), f), jax.ShapeDtypeStruct((1, 256), f),
        ],
        compiler_params=_CP,
    )(nodes_0, wc3, bc3, nodes_1, wc2, bc2, nodes_2, wc1, bc1, nodes_3, wc0, bc0)
    t1, ty1, tz1, tw1, s3, q3, s2, q2, sz, qz, s0, q0 = s1_out

    a1, sh1 = _bn_ab(s3, q3, _C0, cnt=cnt0)

    t3r, s31, q31 = pl.pallas_call(
        _stage2,
        grid=(_STEPS,),
        in_specs=[_bspec(_C0, 256), _cspec(_C0, 256), _cspec(_C0, 256),
                  _cspec(256, 512), _cspec(1, 512)],
        out_specs=[_bspec(_C1, 512), _cspec(_C0, 2), _cspec(_C0, 2)],
        out_shape=[
            jax.ShapeDtypeStruct((_B, _C1, 512), _BF),
            jax.ShapeDtypeStruct((_C0, 2), f), jax.ShapeDtypeStruct((_C0, 2), f),
        ],
        compiler_params=_CP,
    )(t1, a1, sh1, W31, b31[None, :])

    a3, sh3 = _bn_ab(s31, q31, _C0, cnt=cnt1, rows=_C1)
    a2, sh2 = _bn_ab(s2, q2, _C1, cnt=cnt1)

    t5, s5, q5, ty2, sy, qy = pl.pallas_call(
        _stage3,
        grid=(_STEPS,),
        in_specs=[_bspec(_C1, 512), _cspec(_C1, 512), _cspec(_C1, 512),
                  _cspec(512, 1024), _cspec(1, 1024),
                  _bspec(_C1, 256), _cspec(_C1, 256), _cspec(_C1, 256),
                  _cspec(256, 512), _cspec(1, 512)],
        out_specs=[_bspec(_C1, 1024), _cspec(_C1, 16), _cspec(_C1, 16),
                   _bspec(_C1, 512), _cspec(_C1, 8), _cspec(_C1, 8)],
        out_shape=[
            jax.ShapeDtypeStruct((_B, _C1, 1024), _BF),
            jax.ShapeDtypeStruct((_C1, 16), f), jax.ShapeDtypeStruct((_C1, 16), f),
            jax.ShapeDtypeStruct((_B, _C1, 512), _BF),
            jax.ShapeDtypeStruct((_C1, 8), f), jax.ShapeDtypeStruct((_C1, 8), f),
        ],
        compiler_params=_CP,
    )(t3r, a3, sh3, W32, b32[None, :], ty1, a2, sh2, W21, b21[None, :])

    a5, sh5 = _bn_ab(s5, q5, _C1)
    c5 = jnp.mean(sh5, axis=0, keepdims=True)
    ayt, shy = _bn_ab(sy, qy, _C1)
    cyt = jnp.mean(shy, axis=0, keepdims=True)
    azt, shz = _bn_ab(sz, qz, _N1)
    czt = jnp.mean(shz, axis=0, keepdims=True)
    a0t, sh0 = _bn_ab(s0, q0, 1)
    c0t = sh0

    out2d = pl.pallas_call(
        _stage4,
        grid=(_STEPS,),
        in_specs=[_bspec(_C1, 1024), _cspec(_C1, 1024), _cspec(1, 1024),
                  _bspec(_C1, 512), _cspec(_C1, 512), _cspec(1, 512),
                  _bspec(_N1, 256), _cspec(_N1, 256), _cspec(1, 256),
                  _bspec(1, 256), _cspec(1, 256), _cspec(1, 256),
                  _cspec(2048, 2048), _cspec(1, 2048)],
        out_specs=pl.BlockSpec((_TB, 2048), lambda i: (i, 0)),
        out_shape=jax.ShapeDtypeStruct((_B, 2048), f),
        compiler_params=_CP,
    )(t5, a5, c5, ty2, ayt, cyt, tz1, azt, czt, tw1, a0t, c0t, W, b[None, :])

    return out2d.reshape(_B, 2048, 1)
